# baseline (speedup 1.0000x reference)
import functools

import jax
import jax.numpy as jnp
from jax import lax
from jax.experimental import pallas as pl
from jax.experimental.pallas import tpu as pltpu
from jax.experimental.pallas import tpu_sc as plsc

N = 10000
E = 320000
D = 128
R = 9
NC, NS, L = 2, 16, 16
NW = NC * NS
EPW = E // NW
K = 80
NCHUNK = EPW // K
RPT = 1000
NSTRIPE = N // RPT

_vmesh = plsc.VectorSubcoreMesh(core_axis_name="c", subcore_axis_name="s")


@functools.partial(
    pl.kernel,
    out_type=jax.ShapeDtypeStruct((NC, 2, R * N), jnp.float32),
    mesh=_vmesh,
    scratch_types=[
        [pltpu.VMEM((K,), jnp.int32)] * 2,
        [pltpu.VMEM((K,), jnp.int32)] * 2,
        [pltpu.VMEM((K,), jnp.int32)] * 2,
        [pltpu.VMEM((K,), jnp.int32)] * 2,
        [pltpu.VMEM((K,), jnp.int32)] * 2,
        pltpu.VMEM((K,), jnp.float32),
        [pltpu.SemaphoreType.DMA] * 2,
        [pltpu.SemaphoreType.DMA] * 2,
        [pltpu.SemaphoreType.DMA] * 2,
        pltpu.VMEM_SHARED((R * N,), jnp.float32),
        pltpu.VMEM_SHARED((R * N,), jnp.float32),
    ],
)
def _sc_degrees(src_hbm, dst_hbm, et_hbm, z_hbm, out_hbm,
                s_v, d_v, t_v, io_v, ii_v, ones_v,
                sem_a, sem_o, sem_i, degO, degI):
    cid = lax.axis_index("c")
    sid = lax.axis_index("s")
    wid = sid * NC + cid
    for j in range(K // L):
        ones_v[pl.ds(j * L, L)] = jnp.full((L,), 1.0, jnp.float32)

    @pl.when(sid == 0)
    def _():
        pltpu.sync_copy(z_hbm, degO)
        pltpu.sync_copy(z_hbm, degI)

    def load_raw(g, p):
        base = jnp.minimum(wid * EPW + g * K, E - K)
        pltpu.async_copy(src_hbm.at[pl.ds(base, K)], s_v[p], sem_a[p])
        pltpu.async_copy(dst_hbm.at[pl.ds(base, K)], d_v[p], sem_a[p])
        pltpu.async_copy(et_hbm.at[pl.ds(base, K)], t_v[p], sem_a[p])

    def wait_raw(p):
        pltpu.make_async_copy(src_hbm.at[pl.ds(0, K)], s_v[p], sem_a[p]).wait()
        pltpu.make_async_copy(dst_hbm.at[pl.ds(0, K)], d_v[p], sem_a[p]).wait()
        pltpu.make_async_copy(et_hbm.at[pl.ds(0, K)], t_v[p], sem_a[p]).wait()

    def compute_idx(p):
        for j in range(K // L):
            sl = pl.ds(j * L, L)
            tn = t_v[p][sl] * N
            io_v[p][sl] = tn + s_v[p][sl]
            ii_v[p][sl] = tn + d_v[p][sl]

    def start_scatter(p):
        pltpu.async_copy(ones_v, degO.at[io_v[p]], sem_o[p], add=True)
        pltpu.async_copy(ones_v, degI.at[ii_v[p]], sem_i[p], add=True)

    def wait_scatter(p):
        pltpu.make_async_copy(ones_v, degO.at[io_v[p]], sem_o[p]).wait()
        pltpu.make_async_copy(ones_v, degI.at[ii_v[p]], sem_i[p]).wait()

    plsc.subcore_barrier()

    def body(g, p, first):
        wait_raw(p)
        if not first:
            wait_scatter(p)
        compute_idx(p)
        start_scatter(p)
        load_raw(g + 2, p)

    load_raw(0, 0)
    load_raw(1, 1)
    body(0, 0, True)
    body(1, 1, True)

    def pair(i2, carry):
        body(i2 * 2, 0, False)
        body(i2 * 2 + 1, 1, False)
        return carry

    lax.fori_loop(1, (NCHUNK - 1) // 2, pair, 0)
    body(NCHUNK - 1, 0, False)
    wait_scatter(1)
    wait_scatter(0)
    wait_raw(1)
    wait_raw(0)

    plsc.subcore_barrier()

    @pl.when(sid == 0)
    def _():
        pltpu.sync_copy(degO, out_hbm.at[cid, 0])
        pltpu.sync_copy(degI, out_hbm.at[cid, 1])


def _norms_body(degs_ref, ns1_ref, nd1_ref, ns2_ref, nd2_ref):
    dO = degs_ref[0, 0] + degs_ref[1, 0]
    dI = degs_ref[0, 1] + degs_ref[1, 1]
    nsb = jnp.where(dO > 0, lax.rsqrt(jnp.maximum(dO, 1.0)), 0.0)
    ndb = jnp.where(dI > 0, lax.rsqrt(jnp.maximum(dI, 1.0)), 0.0)
    row = lax.broadcasted_iota(jnp.int32, (R, N), 0)
    first6 = row < 6
    ones = jnp.ones((R, N), jnp.float32)
    ns1_ref[...] = jnp.where(first6, ones, nsb)
    nd1_ref[...] = jnp.where(first6, ones, ndb)
    ns2_ref[...] = nsb
    nd2_ref[...] = ndb


def _tc_norms(degs):
    out = jax.ShapeDtypeStruct((R, N), jnp.float32)
    return pl.pallas_call(
        _norms_body,
        out_shape=(out, out, out, out),
    )(degs)


NB_MM = 1024
NBLK_MM = (N + NB_MM - 1) // NB_MM


def _mm_body(x_ref, w_ref, ns_ref, out_ref):
    r = pl.program_id(1)
    xw = jnp.dot(x_ref[...], w_ref[0], preferred_element_type=jnp.float32)
    row = lax.broadcasted_iota(jnp.int32, (R, NB_MM), 0)
    nsr = jnp.sum(jnp.where(row == r, ns_ref[...], 0.0), axis=0)
    out_ref[0] = xw * nsr[:, None]


def _tc_matmul(x, W, ns):
    return pl.pallas_call(
        _mm_body,
        grid=(NBLK_MM, R),
        in_specs=[
            pl.BlockSpec((NB_MM, D), lambda nb, r: (nb, 0)),
            pl.BlockSpec((1, D, D), lambda nb, r: (r, 0, 0)),
            pl.BlockSpec((R, NB_MM), lambda nb, r: (0, nb)),
        ],
        out_specs=pl.BlockSpec((1, NB_MM, D), lambda nb, r: (r, nb, 0)),
        out_shape=jax.ShapeDtypeStruct((R, N, D), jnp.float32),
    )(x, W, ns)


@functools.partial(
    pl.kernel,
    out_type=jax.ShapeDtypeStruct((NC, N, D), jnp.float32),
    mesh=_vmesh,
    scratch_types=[
        [pltpu.VMEM((K,), jnp.int32)] * 2,
        [pltpu.VMEM((K,), jnp.int32)] * 2,
        [pltpu.VMEM((K,), jnp.int32)] * 2,
        [pltpu.VMEM((K,), jnp.int32)] * 2,
        [pltpu.VMEM((K,), jnp.int32)] * 2,
        [pltpu.VMEM((K,), jnp.int32)] * 2,
        [pltpu.VMEM((K,), jnp.float32)] * 2,
        [pltpu.VMEM((K, D), jnp.float32)] * 2,
        [pltpu.SemaphoreType.DMA] * 2,
        [pltpu.SemaphoreType.DMA] * 2,
        [pltpu.SemaphoreType.DMA] * 2,
        [pltpu.SemaphoreType.DMA] * 2,
        pltpu.VMEM_SHARED((N, D), jnp.float32),
    ],
)
def _sc_edges(src_hbm, dst_hbm, et_hbm, xw_hbm, nd_hbm, z_hbm, out_hbm,
              s_v, d_v, t_v, g_v, ci_v, dd_v, c_v, rows_v,
              sem_a, sem_g, sem_c, sem_e, acc):
    cid = lax.axis_index("c")
    sid = lax.axis_index("s")
    wid = sid * NC + cid
    rsl = pl.ds(sid * RPT, RPT)

    @pl.when(sid < NSTRIPE)
    def _():
        pltpu.sync_copy(z_hbm.at[rsl], acc.at[rsl])

    def load_raw(g, p):
        base = jnp.minimum(wid * EPW + g * K, E - K)
        pltpu.async_copy(src_hbm.at[pl.ds(base, K)], s_v[p], sem_a[p])
        pltpu.async_copy(dst_hbm.at[pl.ds(base, K)], d_v[p], sem_a[p])
        pltpu.async_copy(et_hbm.at[pl.ds(base, K)], t_v[p], sem_a[p])

    def wait_raw(p):
        pltpu.make_async_copy(src_hbm.at[pl.ds(0, K)], s_v[p], sem_a[p]).wait()
        pltpu.make_async_copy(dst_hbm.at[pl.ds(0, K)], d_v[p], sem_a[p]).wait()
        pltpu.make_async_copy(et_hbm.at[pl.ds(0, K)], t_v[p], sem_a[p]).wait()

    def compute_idx(p):
        for j in range(K // L):
            sl = pl.ds(j * L, L)
            tn = t_v[p][sl] * N
            g_v[p][sl] = tn + s_v[p][sl]
            ci_v[p][sl] = tn + d_v[p][sl]
            dd_v[p][sl] = d_v[p][sl]

    def start_gather(p):
        h = K // 2
        pltpu.async_copy(xw_hbm.at[g_v[p].at[pl.ds(0, h)]],
                         rows_v[p].at[pl.ds(0, h)], sem_g[p])
        pltpu.async_copy(xw_hbm.at[g_v[p].at[pl.ds(h, h)]],
                         rows_v[p].at[pl.ds(h, h)], sem_g[p])
        pltpu.async_copy(nd_hbm.at[ci_v[p]], c_v[p], sem_c[p])

    def wait_gather(p):
        h = K // 2
        pltpu.make_async_copy(xw_hbm.at[g_v[p].at[pl.ds(0, h)]],
                              rows_v[p].at[pl.ds(0, h)], sem_g[p]).wait()
        pltpu.make_async_copy(xw_hbm.at[g_v[p].at[pl.ds(h, h)]],
                              rows_v[p].at[pl.ds(h, h)], sem_g[p]).wait()
        pltpu.make_async_copy(nd_hbm.at[ci_v[p]], c_v[p], sem_c[p]).wait()

    def scale_rows(p):
        def scale(j, carry2):
            c16 = c_v[p][pl.ds(j * L, L)]
            for l in range(L):
                c = c16[l]
                e = j * L + l
                for k in range(D // L):
                    sl2 = pl.ds(k * L, L)
                    rows_v[p][e, sl2] = rows_v[p][e, sl2] * c
            return carry2

        lax.fori_loop(0, K // L, scale, 0)

    def start_scatter(p):
        pltpu.async_copy(rows_v[p], acc.at[dd_v[p]], sem_e[p], add=True)

    def wait_scatter(p):
        pltpu.make_async_copy(rows_v[p], acc.at[dd_v[p]], sem_e[p]).wait()

    plsc.subcore_barrier()

    def body(g, p, first):
        q = 1 - p
        wait_gather(p)
        wait_raw(q)
        if not first:
            wait_scatter(q)
        compute_idx(q)
        start_gather(q)
        scale_rows(p)
        start_scatter(p)
        load_raw(g + 3, q)

    load_raw(0, 0)
    wait_raw(0)
    compute_idx(0)
    start_gather(0)
    load_raw(1, 1)
    load_raw(2, 0)

    body(0, 0, True)
    body(1, 1, False)

    def pair(i2, carry):
        body(i2 * 2, 0, False)
        body(i2 * 2 + 1, 1, False)
        return carry

    lax.fori_loop(1, (NCHUNK - 1) // 2, pair, 0)

    wait_gather(0)
    scale_rows(0)
    wait_scatter(1)
    start_scatter(0)
    wait_raw(1)
    wait_raw(0)
    wait_scatter(0)

    plsc.subcore_barrier()

    @pl.when(sid < NSTRIPE)
    def _():
        pltpu.sync_copy(acc.at[rsl], out_hbm.at[cid, rsl])


NB_CB = 2000
NBLK_CB = N // NB_CB


def _combine_body(part_ref, b_ref, out_ref):
    s = part_ref[0] + part_ref[1] + jnp.sum(b_ref[...], axis=0)[None, :]
    out_ref[...] = jnp.where(s >= 0, s, 0.01 * s)


def _tc_combine(part, b):
    return pl.pallas_call(
        _combine_body,
        grid=(NBLK_CB,),
        in_specs=[
            pl.BlockSpec((NC, NB_CB, D), lambda i: (0, i, 0)),
            pl.BlockSpec((R, D), lambda i: (0, 0)),
        ],
        out_specs=pl.BlockSpec((NB_CB, D), lambda i: (i, 0)),
        out_shape=jax.ShapeDtypeStruct((N, D), jnp.float32),
    )(part, b)


def kernel(x, edge_index, edge_type, W1, b1, W2, b2):
    src = edge_index[0]
    dst = edge_index[1]
    et = edge_type

    z_deg = jnp.zeros((R * N,), jnp.float32)
    z_acc = jnp.zeros((N, D), jnp.float32)

    degs = _sc_degrees(src, dst, et, z_deg)
    ns1, nd1, ns2, nd2 = _tc_norms(degs.reshape(NC, 2, R, N))

    xw1 = _tc_matmul(x, W1, ns1).reshape(R * N, D)
    p1 = _sc_edges(src, dst, et, xw1, nd1.reshape(R * N), z_acc)
    h = _tc_combine(p1, b1)

    xw2 = _tc_matmul(h, W2, ns2).reshape(R * N, D)
    p2 = _sc_edges(src, dst, et, xw2, nd2.reshape(R * N), z_acc)
    return _tc_combine(p2, b2)


# speedup vs baseline: 1.0123x; 1.0123x over previous
import functools

import jax
import jax.numpy as jnp
from jax import lax
from jax.experimental import pallas as pl
from jax.experimental.pallas import tpu as pltpu
from jax.experimental.pallas import tpu_sc as plsc

N = 10000
E = 320000
D = 128
R = 9
NC, NS, L = 2, 16, 16
NW = NC * NS
EPW = E // NW
K = 80
NCHUNK = EPW // K
RPT = 1000
NSTRIPE = N // RPT

_vmesh = plsc.VectorSubcoreMesh(core_axis_name="c", subcore_axis_name="s")


@functools.partial(
    pl.kernel,
    out_type=jax.ShapeDtypeStruct((NC, 2, R * N), jnp.float32),
    mesh=_vmesh,
    scratch_types=[
        [pltpu.VMEM((K,), jnp.int32)] * 2,
        [pltpu.VMEM((K,), jnp.int32)] * 2,
        [pltpu.VMEM((K,), jnp.int32)] * 2,
        [pltpu.VMEM((K,), jnp.int32)] * 2,
        [pltpu.VMEM((K,), jnp.int32)] * 2,
        pltpu.VMEM((K,), jnp.float32),
        [pltpu.SemaphoreType.DMA] * 2,
        [pltpu.SemaphoreType.DMA] * 2,
        [pltpu.SemaphoreType.DMA] * 2,
        pltpu.VMEM_SHARED((R * N,), jnp.float32),
        pltpu.VMEM_SHARED((R * N,), jnp.float32),
    ],
)
def _sc_degrees(src_hbm, dst_hbm, et_hbm, z_hbm, out_hbm,
                s_v, d_v, t_v, io_v, ii_v, ones_v,
                sem_a, sem_o, sem_i, degO, degI):
    cid = lax.axis_index("c")
    sid = lax.axis_index("s")
    wid = sid * NC + cid
    for j in range(K // L):
        ones_v[pl.ds(j * L, L)] = jnp.full((L,), 1.0, jnp.float32)

    @pl.when(sid == 0)
    def _():
        pltpu.sync_copy(z_hbm, degO)
        pltpu.sync_copy(z_hbm, degI)

    def load_raw(g, p):
        base = jnp.minimum(wid * EPW + g * K, E - K)
        pltpu.async_copy(src_hbm.at[pl.ds(base, K)], s_v[p], sem_a[p])
        pltpu.async_copy(dst_hbm.at[pl.ds(base, K)], d_v[p], sem_a[p])
        pltpu.async_copy(et_hbm.at[pl.ds(base, K)], t_v[p], sem_a[p])

    def wait_raw(p):
        pltpu.make_async_copy(src_hbm.at[pl.ds(0, K)], s_v[p], sem_a[p]).wait()
        pltpu.make_async_copy(dst_hbm.at[pl.ds(0, K)], d_v[p], sem_a[p]).wait()
        pltpu.make_async_copy(et_hbm.at[pl.ds(0, K)], t_v[p], sem_a[p]).wait()

    def compute_idx(p):
        for j in range(K // L):
            sl = pl.ds(j * L, L)
            tn = t_v[p][sl] * N
            io_v[p][sl] = tn + s_v[p][sl]
            ii_v[p][sl] = tn + d_v[p][sl]

    def start_scatter(p):
        pltpu.async_copy(ones_v, degO.at[io_v[p]], sem_o[p], add=True)
        pltpu.async_copy(ones_v, degI.at[ii_v[p]], sem_i[p], add=True)

    def wait_scatter(p):
        pltpu.make_async_copy(ones_v, degO.at[io_v[p]], sem_o[p]).wait()
        pltpu.make_async_copy(ones_v, degI.at[ii_v[p]], sem_i[p]).wait()

    plsc.subcore_barrier()

    def body(g, p, first):
        wait_raw(p)
        if not first:
            wait_scatter(p)
        compute_idx(p)
        start_scatter(p)
        load_raw(g + 2, p)

    load_raw(0, 0)
    load_raw(1, 1)
    body(0, 0, True)
    body(1, 1, True)

    def pair(i2, carry):
        body(i2 * 2, 0, False)
        body(i2 * 2 + 1, 1, False)
        return carry

    lax.fori_loop(1, (NCHUNK - 1) // 2, pair, 0)
    body(NCHUNK - 1, 0, False)
    wait_scatter(1)
    wait_scatter(0)
    wait_raw(1)
    wait_raw(0)

    plsc.subcore_barrier()

    @pl.when(sid == 0)
    def _():
        pltpu.sync_copy(degO, out_hbm.at[cid, 0])
        pltpu.sync_copy(degI, out_hbm.at[cid, 1])


def _norms_body(degs_ref, ns1_ref, nd1_ref, ns2_ref, nd2_ref):
    dO = degs_ref[0, 0] + degs_ref[1, 0]
    dI = degs_ref[0, 1] + degs_ref[1, 1]
    nsb = jnp.where(dO > 0, lax.rsqrt(jnp.maximum(dO, 1.0)), 0.0)
    ndb = jnp.where(dI > 0, lax.rsqrt(jnp.maximum(dI, 1.0)), 0.0)
    row = lax.broadcasted_iota(jnp.int32, (R, N), 0)
    first6 = row < 6
    ones = jnp.ones((R, N), jnp.float32)
    ns1_ref[...] = jnp.where(first6, ones, nsb)
    nd1_ref[...] = jnp.where(first6, ones, ndb)
    ns2_ref[...] = nsb
    nd2_ref[...] = ndb


def _tc_norms(degs):
    out = jax.ShapeDtypeStruct((R, N), jnp.float32)
    return pl.pallas_call(
        _norms_body,
        out_shape=(out, out, out, out),
    )(degs)


NB_MM = 1024
NBLK_MM = (N + NB_MM - 1) // NB_MM


def _mm_body(x_ref, w_ref, ns_ref, out_ref):
    r = pl.program_id(1)
    xw = jnp.dot(x_ref[...], w_ref[0], preferred_element_type=jnp.float32)
    row = lax.broadcasted_iota(jnp.int32, (R, NB_MM), 0)
    nsr = jnp.sum(jnp.where(row == r, ns_ref[...], 0.0), axis=0)
    out_ref[0] = xw * nsr[:, None]


def _cmm_body(p_ref, b_ref, w_ref, ns_ref, out_ref):
    r = pl.program_id(1)
    s = p_ref[0] + p_ref[1] + jnp.sum(b_ref[...], axis=0)[None, :]
    h = jnp.where(s >= 0, s, 0.01 * s)
    xw = jnp.dot(h, w_ref[0], preferred_element_type=jnp.float32)
    row = lax.broadcasted_iota(jnp.int32, (R, NB_MM), 0)
    nsr = jnp.sum(jnp.where(row == r, ns_ref[...], 0.0), axis=0)
    out_ref[0] = xw * nsr[:, None]


def _tc_combine_matmul(part, b, W, ns):
    return pl.pallas_call(
        _cmm_body,
        grid=(NBLK_MM, R),
        in_specs=[
            pl.BlockSpec((NC, NB_MM, D), lambda nb, r: (0, nb, 0)),
            pl.BlockSpec((R, D), lambda nb, r: (0, 0)),
            pl.BlockSpec((1, D, D), lambda nb, r: (r, 0, 0)),
            pl.BlockSpec((R, NB_MM), lambda nb, r: (0, nb)),
        ],
        out_specs=pl.BlockSpec((1, NB_MM, D), lambda nb, r: (r, nb, 0)),
        out_shape=jax.ShapeDtypeStruct((R, N, D), jnp.float32),
    )(part, b, W, ns)


def _tc_matmul(x, W, ns):
    return pl.pallas_call(
        _mm_body,
        grid=(NBLK_MM, R),
        in_specs=[
            pl.BlockSpec((NB_MM, D), lambda nb, r: (nb, 0)),
            pl.BlockSpec((1, D, D), lambda nb, r: (r, 0, 0)),
            pl.BlockSpec((R, NB_MM), lambda nb, r: (0, nb)),
        ],
        out_specs=pl.BlockSpec((1, NB_MM, D), lambda nb, r: (r, nb, 0)),
        out_shape=jax.ShapeDtypeStruct((R, N, D), jnp.float32),
    )(x, W, ns)


@functools.partial(
    pl.kernel,
    out_type=jax.ShapeDtypeStruct((NC, N, D), jnp.float32),
    mesh=_vmesh,
    scratch_types=[
        [pltpu.VMEM((K,), jnp.int32)] * 2,
        [pltpu.VMEM((K,), jnp.int32)] * 2,
        [pltpu.VMEM((K,), jnp.int32)] * 2,
        [pltpu.VMEM((K,), jnp.int32)] * 2,
        [pltpu.VMEM((K,), jnp.int32)] * 2,
        [pltpu.VMEM((K,), jnp.int32)] * 2,
        [pltpu.VMEM((K,), jnp.float32)] * 2,
        [pltpu.VMEM((K, D), jnp.float32)] * 2,
        [pltpu.SemaphoreType.DMA] * 2,
        [pltpu.SemaphoreType.DMA] * 2,
        [pltpu.SemaphoreType.DMA] * 2,
        [pltpu.SemaphoreType.DMA] * 2,
        pltpu.VMEM_SHARED((N, D), jnp.float32),
    ],
)
def _sc_edges(src_hbm, dst_hbm, et_hbm, xw_hbm, nd_hbm, z_hbm, out_hbm,
              s_v, d_v, t_v, g_v, ci_v, dd_v, c_v, rows_v,
              sem_a, sem_g, sem_c, sem_e, acc):
    cid = lax.axis_index("c")
    sid = lax.axis_index("s")
    wid = sid * NC + cid
    rsl = pl.ds(sid * RPT, RPT)

    @pl.when(sid < NSTRIPE)
    def _():
        pltpu.sync_copy(z_hbm.at[rsl], acc.at[rsl])

    def load_raw(g, p):
        base = jnp.minimum(wid * EPW + g * K, E - K)
        pltpu.async_copy(src_hbm.at[pl.ds(base, K)], s_v[p], sem_a[p])
        pltpu.async_copy(dst_hbm.at[pl.ds(base, K)], d_v[p], sem_a[p])
        pltpu.async_copy(et_hbm.at[pl.ds(base, K)], t_v[p], sem_a[p])

    def wait_raw(p):
        pltpu.make_async_copy(src_hbm.at[pl.ds(0, K)], s_v[p], sem_a[p]).wait()
        pltpu.make_async_copy(dst_hbm.at[pl.ds(0, K)], d_v[p], sem_a[p]).wait()
        pltpu.make_async_copy(et_hbm.at[pl.ds(0, K)], t_v[p], sem_a[p]).wait()

    def compute_idx(p):
        for j in range(K // L):
            sl = pl.ds(j * L, L)
            tn = t_v[p][sl] * N
            g_v[p][sl] = tn + s_v[p][sl]
            ci_v[p][sl] = tn + d_v[p][sl]
            dd_v[p][sl] = d_v[p][sl]

    def start_gather(p):
        pltpu.async_copy(xw_hbm.at[g_v[p]], rows_v[p], sem_g[p])
        pltpu.async_copy(nd_hbm.at[ci_v[p]], c_v[p], sem_c[p])

    def wait_gather(p):
        pltpu.make_async_copy(xw_hbm.at[g_v[p]], rows_v[p], sem_g[p]).wait()
        pltpu.make_async_copy(nd_hbm.at[ci_v[p]], c_v[p], sem_c[p]).wait()

    def scale_rows(p):
        def scale(j, carry2):
            c16 = c_v[p][pl.ds(j * L, L)]
            for l in range(L):
                c = c16[l]
                e = j * L + l
                for k in range(D // L):
                    sl2 = pl.ds(k * L, L)
                    rows_v[p][e, sl2] = rows_v[p][e, sl2] * c
            return carry2

        lax.fori_loop(0, K // L, scale, 0)

    def start_scatter(p):
        pltpu.async_copy(rows_v[p], acc.at[dd_v[p]], sem_e[p], add=True)

    def wait_scatter(p):
        pltpu.make_async_copy(rows_v[p], acc.at[dd_v[p]], sem_e[p]).wait()

    plsc.subcore_barrier()

    def body(g, p, first):
        q = 1 - p
        wait_gather(p)
        wait_raw(q)
        if not first:
            wait_scatter(q)
        compute_idx(q)
        start_gather(q)
        scale_rows(p)
        start_scatter(p)
        load_raw(g + 3, q)

    load_raw(0, 0)
    wait_raw(0)
    compute_idx(0)
    start_gather(0)
    load_raw(1, 1)
    load_raw(2, 0)

    body(0, 0, True)
    body(1, 1, False)

    def pair(i2, carry):
        body(i2 * 2, 0, False)
        body(i2 * 2 + 1, 1, False)
        return carry

    lax.fori_loop(1, (NCHUNK - 1) // 2, pair, 0)

    wait_gather(0)
    scale_rows(0)
    wait_scatter(1)
    start_scatter(0)
    wait_raw(1)
    wait_raw(0)
    wait_scatter(0)

    plsc.subcore_barrier()

    @pl.when(sid < NSTRIPE)
    def _():
        pltpu.sync_copy(acc.at[rsl], out_hbm.at[cid, rsl])


NB_CB = 2000
NBLK_CB = N // NB_CB


def _combine_body(part_ref, b_ref, out_ref):
    s = part_ref[0] + part_ref[1] + jnp.sum(b_ref[...], axis=0)[None, :]
    out_ref[...] = jnp.where(s >= 0, s, 0.01 * s)


def _tc_combine(part, b):
    return pl.pallas_call(
        _combine_body,
        grid=(NBLK_CB,),
        in_specs=[
            pl.BlockSpec((NC, NB_CB, D), lambda i: (0, i, 0)),
            pl.BlockSpec((R, D), lambda i: (0, 0)),
        ],
        out_specs=pl.BlockSpec((NB_CB, D), lambda i: (i, 0)),
        out_shape=jax.ShapeDtypeStruct((N, D), jnp.float32),
    )(part, b)


def kernel(x, edge_index, edge_type, W1, b1, W2, b2):
    src = edge_index[0]
    dst = edge_index[1]
    et = edge_type

    z_deg = jnp.zeros((R * N,), jnp.float32)
    z_acc = jnp.zeros((N, D), jnp.float32)

    degs = _sc_degrees(src, dst, et, z_deg)
    ns1, nd1, ns2, nd2 = _tc_norms(degs.reshape(NC, 2, R, N))

    xw1 = _tc_matmul(x, W1, ns1).reshape(R * N, D)
    p1 = _sc_edges(src, dst, et, xw1, nd1.reshape(R * N), z_acc)

    xw2 = _tc_combine_matmul(p1, b1, W2, ns2).reshape(R * N, D)
    p2 = _sc_edges(src, dst, et, xw2, nd2.reshape(R * N), z_acc)
    return _tc_combine(p2, b2)
